# baseline (speedup 1.0000x reference)
import functools

import jax
import jax.numpy as jnp
from jax import lax
from jax.experimental import pallas as pl
from jax.experimental.pallas import tpu as pltpu
from jax.experimental.pallas import tpu_sc as plsc

NUM_EMBEDDINGS = 1000000
EMBEDDING_DIM = 64
BATCH = 16384
N_FIELDS = 26

NC = 2
NS = 16
NW = NC * NS

GB = 128
CH = 2
G = GB * CH
B_PER_W = BATCH // NW
HALVES = B_PER_W // G
N_UNITS = N_FIELDS * HALVES
LANES = 16


def _emb_body(idx_hbm, table_hbm, out_hbm, idx_v, rows_v, slab_v,
              sg0, sg1, so0, so1):
    sem_g = [sg0, sg1]
    sem_o = [so0, so1]
    wid = lax.axis_index("s") * NC + lax.axis_index("c")

    def unit_coords(k):
        f = k % N_FIELDS
        half = k // N_FIELDS
        b0 = wid * B_PER_W + half * G
        return f, b0

    def fire(k, s):
        f, b0 = unit_coords(k)
        pltpu.sync_copy(idx_hbm.at[f].at[pl.ds(b0 // GB, CH)], idx_v.at[s])
        for j in range(CH):
            pltpu.async_copy(
                table_hbm.at[idx_v.at[s].at[j]],
                rows_v.at[s].at[pl.ds(j * GB, GB)],
                sem_g[s],
            )

    def wait_gathers(k, s):
        for j in range(CH):
            pltpu.make_async_copy(
                table_hbm.at[idx_v.at[s].at[j]],
                rows_v.at[s].at[pl.ds(j * GB, GB)],
                sem_g[s],
            ).wait()

    def transpose(k, s):
        staged = rows_v.at[s]
        slab = slab_v.at[s]
        row_ids = [
            jnp.full((LANES,), gi * LANES, jnp.int32)
            + lax.iota(jnp.int32, LANES)
            for gi in range(G // LANES)
        ]

        def dcol(d, carry):
            col = jnp.full((LANES,), d, jnp.int32)
            for gi in range(G // LANES):
                vals = plsc.load_gather(staged, [row_ids[gi], col])
                slab[d, pl.ds(gi * LANES, LANES)] = vals
            return carry

        lax.fori_loop(0, EMBEDDING_DIM, dcol, 0)

    def writeback(k, s):
        f, b0 = unit_coords(k)
        pltpu.async_copy(
            slab_v.at[s], out_hbm.at[f].at[:, pl.ds(b0, G)], sem_o[s]
        )

    def wait_writeback(k, s):
        f, b0 = unit_coords(k)
        pltpu.make_async_copy(
            slab_v.at[s], out_hbm.at[f].at[:, pl.ds(b0, G)], sem_o[s]
        ).wait()

    fire(0, 0)
    fire(1, 1)
    for k in (0, 1):
        s = k % 2
        wait_gathers(k, s)
        transpose(k, s)
        fire(k + 2, s)
        writeback(k, s)

    def pair(t, carry):
        for j in range(2):
            k = 2 * t + j
            s = j
            wait_gathers(k, s)
            wait_writeback(k - 2, s)
            transpose(k, s)
            fire(k + 2, s)
            writeback(k, s)
        return carry

    lax.fori_loop(1, N_UNITS // 2 - 1, pair, 0)

    for k in (N_UNITS - 2, N_UNITS - 1):
        s = k % 2
        wait_gathers(k, s)
        wait_writeback(k - 2, s)
        transpose(k, s)
        writeback(k, s)
    for k in (N_UNITS - 2, N_UNITS - 1):
        wait_writeback(k, k % 2)


@functools.partial(jax.jit, static_argnames=())
def kernel(x, embedding_weight):
    idx3 = x.astype(jnp.int32).T.reshape(N_FIELDS, BATCH // GB, GB)
    mesh = plsc.VectorSubcoreMesh(
        core_axis_name="c", subcore_axis_name="s",
        num_cores=NC, num_subcores=NS,
    )
    out3 = pl.kernel(
        _emb_body,
        out_type=jax.ShapeDtypeStruct((N_FIELDS, EMBEDDING_DIM, BATCH),
                                      jnp.float32),
        mesh=mesh,
        scratch_types=[
            pltpu.VMEM((2, CH, GB), jnp.int32),
            pltpu.VMEM((2, G, EMBEDDING_DIM), jnp.float32),
            pltpu.VMEM((2, EMBEDDING_DIM, G), jnp.float32),
            pltpu.SemaphoreType.DMA,
            pltpu.SemaphoreType.DMA,
            pltpu.SemaphoreType.DMA,
            pltpu.SemaphoreType.DMA,
        ],
        compiler_params=pltpu.CompilerParams(
            use_tc_tiling_on_sc=False, needs_layout_passes=False),
    )(idx3, embedding_weight)
    return jnp.transpose(out3, (2, 0, 1))


# speedup vs baseline: 1.4329x; 1.4329x over previous
import functools

import jax
import jax.numpy as jnp
from jax import lax
from jax.experimental import pallas as pl
from jax.experimental.pallas import tpu as pltpu
from jax.experimental.pallas import tpu_sc as plsc

NUM_EMBEDDINGS = 1000000
EMBEDDING_DIM = 64
BATCH = 16384
N_FIELDS = 26

NC = 2
NS = 16
NW = NC * NS

GB = 128
CH = 2
G = GB * CH
B_PER_W = BATCH // NW
HALVES = B_PER_W // G
N_UNITS = N_FIELDS * HALVES
NBUF = 3


def _emb_body(idx_hbm, table_hbm, out_hbm, idx_v, rows_v,
              sg0, sg1, sg2, so0, so1, so2):
    sem_g = [sg0, sg1, sg2]
    sem_o = [so0, so1, so2]
    wid = lax.axis_index("s") * NC + lax.axis_index("c")

    def unit_coords(k):
        f = k % N_FIELDS
        half = k // N_FIELDS
        b0 = wid * B_PER_W + half * G
        return f, b0

    def fire(k, s):
        f, b0 = unit_coords(k)
        for j in range(CH):
            pltpu.sync_copy(idx_hbm.at[f, pl.ds(b0 + j * GB, GB)],
                            idx_v.at[s].at[j])
        for j in range(CH):
            pltpu.async_copy(
                table_hbm.at[idx_v.at[s].at[j]],
                rows_v.at[s].at[pl.ds(j * GB, GB)],
                sem_g[s],
            )

    def wait_gathers(s):
        for j in range(CH):
            pltpu.make_async_copy(
                table_hbm.at[idx_v.at[s].at[j]],
                rows_v.at[s].at[pl.ds(j * GB, GB)],
                sem_g[s],
            ).wait()

    def writeback(k, s):
        f, b0 = unit_coords(k)
        pltpu.async_copy(
            rows_v.at[s], out_hbm.at[pl.ds(b0, G), f], sem_o[s]
        )

    def wait_writeback(k, s):
        f, b0 = unit_coords(k)
        pltpu.make_async_copy(
            rows_v.at[s], out_hbm.at[pl.ds(b0, G), f], sem_o[s]
        ).wait()

    fire(0, 0)
    fire(1, 1)
    wait_gathers(0)
    writeback(0, 0)
    fire(2, 2)
    wait_gathers(1)
    writeback(1, 1)

    def triple(t, carry):
        for j in range(NBUF):
            k = 3 * t + j
            s = j
            sp = (j + NBUF - 1) % NBUF
            wait_writeback(k - NBUF, s)
            fire(k, s)
            wait_gathers(sp)
            writeback(k - 1, sp)
        return carry

    lax.fori_loop(1, 17, triple, 0)

    k = N_UNITS - 1
    wait_writeback(k - NBUF, 0)
    fire(k, 0)
    wait_gathers(2)
    writeback(k - 1, 2)
    wait_gathers(0)
    writeback(k, 0)
    wait_writeback(N_UNITS - 3, 1)
    wait_writeback(N_UNITS - 2, 2)
    wait_writeback(N_UNITS - 1, 0)


@functools.partial(jax.jit, static_argnames=())
def kernel(x, embedding_weight):
    idx_t = x.astype(jnp.int32).T
    mesh = plsc.VectorSubcoreMesh(
        core_axis_name="c", subcore_axis_name="s",
        num_cores=NC, num_subcores=NS,
    )
    out = pl.kernel(
        _emb_body,
        out_type=jax.ShapeDtypeStruct((BATCH, N_FIELDS, EMBEDDING_DIM),
                                      jnp.float32),
        mesh=mesh,
        scratch_types=[
            pltpu.VMEM((NBUF, CH, GB), jnp.int32),
            pltpu.VMEM((NBUF, G, EMBEDDING_DIM), jnp.float32),
            pltpu.SemaphoreType.DMA,
            pltpu.SemaphoreType.DMA,
            pltpu.SemaphoreType.DMA,
            pltpu.SemaphoreType.DMA,
            pltpu.SemaphoreType.DMA,
            pltpu.SemaphoreType.DMA,
        ],
        compiler_params=pltpu.CompilerParams(
            use_tc_tiling_on_sc=False, needs_layout_passes=False),
    )(idx_t, embedding_weight)
    return out
